# baseline (speedup 1.0000x reference)
import functools

import jax
import jax.numpy as jnp
from jax import lax
from jax.experimental import pallas as pl
from jax.experimental.pallas import tpu as pltpu
from jax.experimental.pallas import tpu_sc as plsc

_B = 16384
_D = 64
_LANES = 16
_CHUNK = 128

_NC = 2
_NS = 16
_NW = _NC * _NS
_BPW = _B // _NW
_NCHUNK = _BPW // _CHUNK
_NGROUP = _BPW // _LANES


def _bmf_body(uid_hbm, iid_hbm, ut_hbm, it_hbm, ub_hbm, ib_hbm, gb_hbm,
              out_hbm,
              uidx_v, iidx_v, uq_v, iq_v, urows_v, irows_v, ubias_v, ibias_v,
              out_v, gb_v, sem):
    wid = lax.axis_index("s") * _NC + lax.axis_index("c")
    cbase = wid * _NCHUNK
    base = wid * _BPW

    pltpu.sync_copy(uid_hbm.at[pl.ds(cbase, _NCHUNK)], uidx_v)
    pltpu.sync_copy(iid_hbm.at[pl.ds(cbase, _NCHUNK)], iidx_v)
    pltpu.sync_copy(gb_hbm, gb_v)

    for j in range(_NCHUNK):
        for k in range(_CHUNK // _LANES):
            sl = pl.ds(k * _LANES, _LANES)
            uq_v[j, sl] = uidx_v[j, sl] >> 4
            iq_v[j, sl] = iidx_v[j, sl] >> 4

    copies = []
    for j in range(_NCHUNK):
        s = j * _CHUNK
        copies.append(pltpu.async_copy(
            ut_hbm.at[uidx_v.at[j]], urows_v.at[pl.ds(s, _CHUNK)], sem))
        copies.append(pltpu.async_copy(
            it_hbm.at[iidx_v.at[j]], irows_v.at[pl.ds(s, _CHUNK)], sem))
        copies.append(pltpu.async_copy(
            ub_hbm.at[uq_v.at[j]], ubias_v.at[pl.ds(s, _CHUNK)], sem))
        copies.append(pltpu.async_copy(
            ib_hbm.at[iq_v.at[j]], ibias_v.at[pl.ds(s, _CHUNK)], sem))
    for c in copies:
        c.wait()

    gb = gb_v[...]

    def group(g, carry):
        p = g * _LANES + lax.iota(jnp.int32, _LANES)
        acc = jnp.zeros((_LANES,), jnp.float32)
        for d in range(_D):
            col = jnp.full((_LANES,), d, jnp.int32)
            u = plsc.load_gather(urows_v, [p, col])
            v = plsc.load_gather(irows_v, [p, col])
            acc = acc + u * v
        jv = p >> 7
        kv = p & 127
        uidx = plsc.load_gather(uidx_v, [jv, kv])
        iidx = plsc.load_gather(iidx_v, [jv, kv])
        ub = plsc.load_gather(ubias_v, [p, uidx & 15])
        ib = plsc.load_gather(ibias_v, [p, iidx & 15])
        z = acc + ub + ib + gb
        out_v[pl.ds(g * _LANES, _LANES)] = 1.0 / (1.0 + jnp.exp(-z))
        return carry

    lax.fori_loop(0, _NGROUP, group, 0)
    pltpu.sync_copy(out_v, out_hbm.at[pl.ds(base, _BPW)])


@jax.jit
def _bmf(uid, iid, ut, it, ub, ib, gb):
    mesh = plsc.VectorSubcoreMesh(core_axis_name="c", subcore_axis_name="s")
    kfn = pl.kernel(
        _bmf_body,
        mesh=mesh,
        compiler_params=pltpu.CompilerParams(
            needs_layout_passes=False, use_tc_tiling_on_sc=False),
        out_type=jax.ShapeDtypeStruct((_B,), jnp.float32),
        scratch_types=[
            pltpu.VMEM((_NCHUNK, _CHUNK), jnp.int32),
            pltpu.VMEM((_NCHUNK, _CHUNK), jnp.int32),
            pltpu.VMEM((_NCHUNK, _CHUNK), jnp.int32),
            pltpu.VMEM((_NCHUNK, _CHUNK), jnp.int32),
            pltpu.VMEM((_BPW, _D), jnp.float32),
            pltpu.VMEM((_BPW, _D), jnp.float32),
            pltpu.VMEM((_BPW, _LANES), jnp.float32),
            pltpu.VMEM((_BPW, _LANES), jnp.float32),
            pltpu.VMEM((_BPW,), jnp.float32),
            pltpu.VMEM((_LANES,), jnp.float32),
            pltpu.SemaphoreType.DMA,
        ],
    )
    return kfn(uid, iid, ut, it, ub, ib, gb)


def kernel(user_ids, item_ids, user_table, item_table, user_bias_table,
           item_bias_table, global_bias):
    uid = user_ids.astype(jnp.int32).reshape(_B // _CHUNK, _CHUNK)
    iid = item_ids.astype(jnp.int32).reshape(_B // _CHUNK, _CHUNK)
    ubq = user_bias_table.reshape(-1, _LANES)
    ibq = item_bias_table.reshape(-1, _LANES)
    gb = jnp.broadcast_to(global_bias.reshape(()), (_LANES,))
    out = _bmf(uid, iid, user_table, item_table, ubq, ibq, gb)
    return out.reshape(_B, 1)
